# baseline (speedup 1.0000x reference)
import functools

import jax
import jax.numpy as jnp
from jax import lax
from jax.experimental import pallas as pl

B, N, D, K = 64, 576, 64, 512
ROWS = B * N
BLK = 512
GRID = ROWS // BLK


def _body(x_ref, mask_ref, ct_ref, mean_ref, scale_ref, out_ref):
    x = x_ref[...]
    ct = ct_ref[...]
    mean = mean_ref[...]
    scale = scale_ref[...]
    xs = (x - mean) / scale
    b2 = jnp.sum(ct * ct, axis=0, keepdims=True)
    ab = jnp.dot(xs, ct, preferred_element_type=jnp.float32)
    scores = b2 - 2.0 * ab
    mn = jnp.min(scores, axis=1, keepdims=True)
    iota = lax.broadcasted_iota(jnp.int32, (BLK, K), 1)
    cand = jnp.where(scores == mn, iota, K)
    first = jnp.min(cand, axis=1, keepdims=True)
    onehot = (iota == first).astype(jnp.float32)
    out_ref[...] = onehot * mask_ref[...]


@jax.jit
def kernel(X, mask, centroids, mean, scale):
    x2 = X.reshape(ROWS, D)
    m2 = mask.reshape(ROWS, 1)
    out = pl.pallas_call(
        _body,
        grid=(GRID,),
        in_specs=[
            pl.BlockSpec((BLK, D), lambda i: (i, 0)),
            pl.BlockSpec((BLK, 1), lambda i: (i, 0)),
            pl.BlockSpec((D, K), lambda i: (0, 0)),
            pl.BlockSpec((1, D), lambda i: (0, 0)),
            pl.BlockSpec((1, D), lambda i: (0, 0)),
        ],
        out_specs=pl.BlockSpec((BLK, K), lambda i: (i, 0)),
        out_shape=jax.ShapeDtypeStruct((ROWS, K), jnp.float32),
    )(x2, m2, centroids.T, mean.reshape(1, D), scale.reshape(1, D))
    return out.reshape(B, N, K)


# speedup vs baseline: 1.3162x; 1.3162x over previous
import functools

import jax
import jax.numpy as jnp
from jax import lax
from jax.experimental import pallas as pl
from jax.experimental.pallas import tpu as pltpu

B, N, D, K = 64, 576, 64, 512


def _body(x_ref, mask_ref, c_ref, mean_ref, scale_ref, out_ref, ct_ref, b2_ref):
    @pl.when(pl.program_id(0) == 0)
    def _init():
        ct = c_ref[...].T
        ct_ref[...] = ct
        b2_ref[...] = jnp.sum(ct * ct, axis=0, keepdims=True)

    x = x_ref[0]
    mean = mean_ref[...]
    scale = scale_ref[...]
    xs = (x - mean) / scale
    ab = jnp.dot(xs, ct_ref[...], preferred_element_type=jnp.float32)
    scores = b2_ref[...] - 2.0 * ab
    mn = jnp.min(scores, axis=1, keepdims=True)
    iota = lax.broadcasted_iota(jnp.int32, (N, K), 1)
    cand = jnp.where(scores == mn, iota, K)
    first = jnp.min(cand, axis=1, keepdims=True)
    onehot = (iota == first).astype(jnp.float32)
    mask_col = mask_ref[0, 0][:, None]
    out_ref[0] = onehot * mask_col


@jax.jit
def kernel(X, mask, centroids, mean, scale):
    return pl.pallas_call(
        _body,
        grid=(B,),
        in_specs=[
            pl.BlockSpec((1, N, D), lambda i: (i, 0, 0)),
            pl.BlockSpec((1, 1, N), lambda i: (i, 0, 0)),
            pl.BlockSpec((K, D), lambda i: (0, 0)),
            pl.BlockSpec((1, D), lambda i: (0, 0)),
            pl.BlockSpec((1, D), lambda i: (0, 0)),
        ],
        out_specs=pl.BlockSpec((1, N, K), lambda i: (i, 0, 0)),
        out_shape=jax.ShapeDtypeStruct((B, N, K), jnp.float32),
        scratch_shapes=[
            pltpu.VMEM((D, K), jnp.float32),
            pltpu.VMEM((1, K), jnp.float32),
        ],
    )(X, mask.reshape(B, 1, N), centroids, mean.reshape(1, D), scale.reshape(1, D))
